# baseline (speedup 1.0000x reference)
import functools

import jax
import jax.numpy as jnp
from jax import lax
from jax.experimental import pallas as pl
from jax.experimental.pallas import tpu as pltpu
from jax.experimental.pallas import tpu_sc as plsc

_NUM_VIEWS = 6
_VOCAB = 64
_BATCH = 16384
_NC = 2
_NS = 16
_NW = _NC * _NS
_LANES = 16
_BPW = _BATCH // _NW
_GROUPS = _BPW // _LANES


def _make_sc_gather():
    mesh = plsc.VectorSubcoreMesh(core_axis_name="c", subcore_axis_name="s",
                                  num_cores=_NC)

    @functools.partial(
        pl.kernel,
        mesh=mesh,
        compiler_params=pltpu.CompilerParams(needs_layout_passes=False),
        out_type=jax.ShapeDtypeStruct((_NUM_VIEWS, _BATCH), jnp.float32),
        scratch_types=[
            pltpu.VMEM((_BPW,), jnp.int32),
            pltpu.VMEM((_VOCAB * _NUM_VIEWS,), jnp.float32),
            pltpu.VMEM((_NUM_VIEWS, _BPW), jnp.float32),
            pltpu.SemaphoreType.DMA,
            pltpu.SemaphoreType.DMA,
        ],
    )
    def gather_kernel(idx_hbm, tab_hbm, out_hbm, idx_v, tab_v, out_v,
                      sem_tab, sem_idx):
        wid = lax.axis_index("s") * _NC + lax.axis_index("c")
        base = wid * _BPW
        cp_tab = pltpu.async_copy(tab_hbm, tab_v, sem_tab)
        cp_idx = pltpu.async_copy(idx_hbm.at[pl.ds(base, _BPW)], idx_v,
                                  sem_idx)
        cp_tab.wait()
        cp_idx.wait()

        @plsc.parallel_loop(0, _GROUPS, 1, unroll=2)
        def _loop(g):
            off = g * _LANES
            ids = idx_v[pl.ds(off, _LANES)]
            src = ids * _NUM_VIEWS
            for d in range(_NUM_VIEWS):
                vals = plsc.load_gather(tab_v, [src + d])
                out_v[d, pl.ds(off, _LANES)] = vals
        pltpu.sync_copy(out_v, out_hbm.at[:, pl.ds(base, _BPW)])

    return gather_kernel


_SC_GATHER = _make_sc_gather()


def kernel(missing_pattern, tag_table):
    tags_t = _SC_GATHER(missing_pattern.astype(jnp.int32),
                        tag_table.reshape(-1))
    return tags_t.T


# speedup vs baseline: 1.0873x; 1.0873x over previous
import functools

import jax
import jax.numpy as jnp
from jax import lax
from jax.experimental import pallas as pl
from jax.experimental.pallas import tpu as pltpu
from jax.experimental.pallas import tpu_sc as plsc

_NUM_VIEWS = 6
_VOCAB = 64
_BATCH = 16384
_NC = 1
_NS = 16
_NW = _NC * _NS
_LANES = 16
_BPW = _BATCH // _NW
_GROUPS = _BPW // _LANES


def _make_sc_gather():
    mesh = plsc.VectorSubcoreMesh(core_axis_name="c", subcore_axis_name="s",
                                  num_cores=_NC)

    @functools.partial(
        pl.kernel,
        mesh=mesh,
        compiler_params=pltpu.CompilerParams(needs_layout_passes=False),
        out_type=jax.ShapeDtypeStruct((_NUM_VIEWS, _BATCH), jnp.float32),
        scratch_types=[
            pltpu.VMEM((_BPW,), jnp.int32),
            pltpu.VMEM((_VOCAB * _NUM_VIEWS,), jnp.float32),
            pltpu.VMEM((_NUM_VIEWS, _BPW), jnp.float32),
            pltpu.SemaphoreType.DMA,
            pltpu.SemaphoreType.DMA,
        ],
    )
    def gather_kernel(idx_hbm, tab_hbm, out_hbm, idx_v, tab_v, out_v,
                      sem_tab, sem_idx):
        wid = lax.axis_index("s") * _NC + lax.axis_index("c")
        base = wid * _BPW
        cp_tab = pltpu.async_copy(tab_hbm, tab_v, sem_tab)
        cp_idx = pltpu.async_copy(idx_hbm.at[pl.ds(base, _BPW)], idx_v,
                                  sem_idx)
        cp_tab.wait()
        cp_idx.wait()

        @plsc.parallel_loop(0, _GROUPS, 1, unroll=2)
        def _loop(g):
            off = g * _LANES
            ids = idx_v[pl.ds(off, _LANES)]
            src = ids * _NUM_VIEWS
            for d in range(_NUM_VIEWS):
                vals = plsc.load_gather(tab_v, [src + d])
                out_v[d, pl.ds(off, _LANES)] = vals
        pltpu.sync_copy(out_v, out_hbm.at[:, pl.ds(base, _BPW)])

    return gather_kernel


_SC_GATHER = _make_sc_gather()


def kernel(missing_pattern, tag_table):
    tags_t = _SC_GATHER(missing_pattern.astype(jnp.int32),
                        tag_table.reshape(-1))
    return tags_t.T
